# baseline (speedup 1.0000x reference)
import jax
import jax.numpy as jnp
from jax.experimental import pallas as pl
from jax.experimental.pallas import tpu as pltpu

_BM = 32
_S = 16

_NEG = float("-inf")


def _shift_down(a, s, shape):
    pad = jnp.full(shape, _NEG, jnp.float32)
    return jnp.concatenate([pad, a[..., :a.shape[-2] - s, :]], axis=a.ndim - 2)


def _shift_up(a, s, shape):
    pad = jnp.full(shape, _NEG, jnp.float32)
    return jnp.concatenate([a[..., s:, :], pad], axis=a.ndim - 2)


def _body(xf_ref, xm_ref, yf_ref, w1t_ref, w2t_ref, w3t_ref, vr_ref, w4t_ref,
          wb_ref, out_ref, cn_ref, pre_ref, suf_ref, agg_ref, zb_ref,
          a1_ref):
    j = pl.program_id(1)
    _, L, H = xf_ref.shape
    C = xm_ref.shape[1]
    D = w4t_ref.shape[0]
    S = _S
    NSC = L // S

    xb = xf_ref[0]
    yb = yf_ref[0]
    xm = xm_ref[0]

    @pl.when(j == 0)
    def _():
        cn_ref[:] = (
            jnp.dot(yb, w2t_ref[:], preferred_element_type=jnp.float32)
            + wb_ref[:]
        )
        n_sub = jax.lax.broadcasted_iota(jnp.int32, (L, 1), 0) & (S - 1)
        pre = xb
        suf = xb
        s = 1
        while s < S:
            pre = jnp.where(n_sub >= s, jnp.maximum(pre, _shift_down(pre, s, (s, H))), pre)
            suf = jnp.where(n_sub < S - s, jnp.maximum(suf, _shift_up(suf, s, (s, H))), suf)
            s *= 2
        pre_ref[:] = pre
        suf_ref[:] = suf
        agg_ref[:] = suf.reshape(NSC, S, H)[:, 0, :]
        xv = jnp.dot(xb, vr_ref[:], preferred_element_type=jnp.float32)
        zb_ref[:] = jax.lax.dot_general(
            xv.reshape(L * D, H), yb, (((1,), (1,)), ((), ())),
            preferred_element_type=jnp.float32).reshape(L, D, L)
        a1_ref[:] = jnp.dot(xb, w1t_ref[:], preferred_element_type=jnp.float32)

    c_ids = jax.lax.broadcasted_iota(jnp.int32, (NSC, 1), 0)
    agg = agg_ref[:]
    pre = pre_ref[:]
    suf = suf_ref[:]
    n3 = jax.lax.broadcasted_iota(jnp.int32, (1, L, 1), 1)
    nsc = n3 // S
    mi3 = jax.lax.broadcasted_iota(jnp.int32, (S, S, 1), 0)
    ni3 = jax.lax.broadcasted_iota(jnp.int32, (S, S, 1), 1)
    ge = ni3 >= mi3
    le = ni3 <= mi3
    ctx_groups = []
    for g in range(C // S):
        scg = j * (C // S) + g
        up = jnp.where(c_ids > scg, agg, _NEG)
        dn = jnp.where(c_ids < scg, agg, _NEG)
        s = 1
        while s < NSC:
            up = jnp.maximum(up, _shift_down(up, s, (s, H)))
            dn = jnp.maximum(dn, _shift_up(dn, s, (s, H)))
            s *= 2
        up = _shift_down(up, 1, (1, H))
        dn = _shift_up(dn, 1, (1, H))
        gup = jnp.maximum(
            jnp.broadcast_to(up.reshape(NSC, 1, H),
                             (NSC, S, H)).reshape(L, H), pre)
        gdn = jnp.maximum(
            jnp.broadcast_to(dn.reshape(NSC, 1, H),
                             (NSC, S, H)).reshape(L, H), suf)

        m0 = j * C + g * S
        suf_g = suf_ref[pl.ds(m0, S), :]
        pre_g = pre_ref[pl.ds(m0, S), :]
        rowup = jnp.maximum(suf_g[:, None, :], gup[None])
        rowdn = jnp.maximum(pre_g[:, None, :], gdn[None])
        base = jnp.where(n3 < scg * S, rowdn, rowup)

        xg = xm[g * S:(g + 1) * S]
        fwd = jnp.where(ge, xg[None], _NEG)
        bwd = jnp.where(le, xg[None], _NEG)
        s = 1
        while s < S:
            fwd = jnp.maximum(fwd, _shift_down(fwd, s, (S, s, H)))
            bwd = jnp.maximum(bwd, _shift_up(bwd, s, (S, s, H)))
            s *= 2
        tri = jnp.where(ge, fwd, bwd)
        tri_exp = jnp.broadcast_to(tri.reshape(S, 1, S, H),
                                   (S, NSC, S, H)).reshape(S, L, H)
        ctx_groups.append(jnp.where(nsc == scg, tri_exp, base))

    ctx = jnp.concatenate(ctx_groups, axis=0)

    s3 = jnp.dot(ctx.reshape(C * L, H), w3t_ref[:],
                 preferred_element_type=jnp.float32).reshape(C, L, H)

    cn = cn_ref[:]
    half = jnp.float32(0.5)
    isq2 = jnp.float32(0.7071067811865476)
    for mi in range(C):
        z_mi = jnp.transpose(zb_ref[pl.ds(j * C + mi, 1)][0], (1, 0))
        t4_mi = jnp.dot(z_mi, w4t_ref[:], preferred_element_type=jnp.float32)
        acc = (s3[mi] + t4_mi) + (cn + a1_ref[pl.ds(j * C + mi, 1)])
        out_ref[0, mi] = half * acc * (1.0 + jax.lax.erf(acc * isq2))


def kernel(x, y, V, W_w, W_b):
    B, L, H = x.shape
    D = V.shape[0]
    BM = _BM

    w1t = W_w[:, :H].T
    w2t = W_w[:, H:2 * H].T
    w3t = W_w[:, 2 * H:3 * H].T
    w4t = W_w[:, 3 * H:].T
    vr = jnp.transpose(V, (1, 0, 2)).reshape(H, D * H)
    wb2 = W_b[None, :]

    return pl.pallas_call(
        _body,
        grid=(B, L // BM),
        compiler_params=pltpu.CompilerParams(
            dimension_semantics=("parallel", "arbitrary")),
        in_specs=[
            pl.BlockSpec((1, L, H), lambda b, j: (b, 0, 0)),
            pl.BlockSpec((1, BM, H), lambda b, j: (b, j, 0)),
            pl.BlockSpec((1, L, H), lambda b, j: (b, 0, 0)),
            pl.BlockSpec((H, H), lambda b, j: (0, 0)),
            pl.BlockSpec((H, H), lambda b, j: (0, 0)),
            pl.BlockSpec((H, H), lambda b, j: (0, 0)),
            pl.BlockSpec((H, D * H), lambda b, j: (0, 0)),
            pl.BlockSpec((D, H), lambda b, j: (0, 0)),
            pl.BlockSpec((1, H), lambda b, j: (0, 0)),
        ],
        out_specs=pl.BlockSpec((1, BM, L, H), lambda b, j: (b, j, 0, 0)),
        out_shape=jax.ShapeDtypeStruct((B, L, L, H), jnp.float32),
        scratch_shapes=[
            pltpu.VMEM((L, H), jnp.float32),
            pltpu.VMEM((L, H), jnp.float32),
            pltpu.VMEM((L, H), jnp.float32),
            pltpu.VMEM((L // _S, H), jnp.float32),
            pltpu.VMEM((L, D, L), jnp.float32),
            pltpu.VMEM((L, H), jnp.float32),
        ],
    )(x, x, y, w1t, w2t, w3t, vr, w4t, wb2)


# speedup vs baseline: 1.0240x; 1.0240x over previous
import jax
import jax.numpy as jnp
from jax.experimental import pallas as pl
from jax.experimental.pallas import tpu as pltpu

_BM = 32
_S = 32

_NEG = float("-inf")


def _shift_down(a, s, shape):
    pad = jnp.full(shape, _NEG, jnp.float32)
    return jnp.concatenate([pad, a[..., :a.shape[-2] - s, :]], axis=a.ndim - 2)


def _shift_up(a, s, shape):
    pad = jnp.full(shape, _NEG, jnp.float32)
    return jnp.concatenate([a[..., s:, :], pad], axis=a.ndim - 2)


def _body(xf_ref, xm_ref, yf_ref, w1t_ref, w2t_ref, w3t_ref, vr_ref, w4t_ref,
          wb_ref, out_ref, cn_ref, pre_ref, suf_ref, agg_ref, zb_ref,
          a1_ref):
    j = pl.program_id(1)
    _, L, H = xf_ref.shape
    C = xm_ref.shape[1]
    D = w4t_ref.shape[0]
    S = _S
    NSC = L // S

    xb = xf_ref[0]
    yb = yf_ref[0]
    xm = xm_ref[0]

    @pl.when(j == 0)
    def _():
        cn_ref[:] = (
            jnp.dot(yb, w2t_ref[:], preferred_element_type=jnp.float32)
            + wb_ref[:]
        )
        n_sub = jax.lax.broadcasted_iota(jnp.int32, (L, 1), 0) & (S - 1)
        pre = xb
        suf = xb
        s = 1
        while s < S:
            pre = jnp.where(n_sub >= s, jnp.maximum(pre, _shift_down(pre, s, (s, H))), pre)
            suf = jnp.where(n_sub < S - s, jnp.maximum(suf, _shift_up(suf, s, (s, H))), suf)
            s *= 2
        pre_ref[:] = pre
        suf_ref[:] = suf
        agg_ref[:] = suf.reshape(NSC, S, H)[:, 0, :]
        xv = jnp.dot(xb, vr_ref[:], preferred_element_type=jnp.float32)
        zb_ref[:] = jax.lax.dot_general(
            xv.reshape(L * D, H), yb, (((1,), (1,)), ((), ())),
            preferred_element_type=jnp.float32).reshape(L, D, L)
        a1_ref[:] = jnp.dot(xb, w1t_ref[:], preferred_element_type=jnp.float32)

    c_ids = jax.lax.broadcasted_iota(jnp.int32, (NSC, 1), 0)
    agg = agg_ref[:]
    pre = pre_ref[:]
    suf = suf_ref[:]
    n3 = jax.lax.broadcasted_iota(jnp.int32, (1, L, 1), 1)
    nsc = n3 // S
    mi3 = jax.lax.broadcasted_iota(jnp.int32, (S, S, 1), 0)
    ni3 = jax.lax.broadcasted_iota(jnp.int32, (S, S, 1), 1)
    ge = ni3 >= mi3
    le = ni3 <= mi3
    ctx_groups = []
    for g in range(C // S):
        scg = j * (C // S) + g
        up = jnp.where(c_ids > scg, agg, _NEG)
        dn = jnp.where(c_ids < scg, agg, _NEG)
        s = 1
        while s < NSC:
            up = jnp.maximum(up, _shift_down(up, s, (s, H)))
            dn = jnp.maximum(dn, _shift_up(dn, s, (s, H)))
            s *= 2
        up = _shift_down(up, 1, (1, H))
        dn = _shift_up(dn, 1, (1, H))
        gup = jnp.maximum(
            jnp.broadcast_to(up.reshape(NSC, 1, H),
                             (NSC, S, H)).reshape(L, H), pre)
        gdn = jnp.maximum(
            jnp.broadcast_to(dn.reshape(NSC, 1, H),
                             (NSC, S, H)).reshape(L, H), suf)

        m0 = j * C + g * S
        suf_g = suf_ref[pl.ds(m0, S), :]
        pre_g = pre_ref[pl.ds(m0, S), :]
        rowup = jnp.maximum(suf_g[:, None, :], gup[None])
        rowdn = jnp.maximum(pre_g[:, None, :], gdn[None])
        base = jnp.where(n3 < scg * S, rowdn, rowup)

        xg = xm[g * S:(g + 1) * S]
        fwd = jnp.where(ge, xg[None], _NEG)
        bwd = jnp.where(le, xg[None], _NEG)
        s = 1
        while s < S:
            fwd = jnp.maximum(fwd, _shift_down(fwd, s, (S, s, H)))
            bwd = jnp.maximum(bwd, _shift_up(bwd, s, (S, s, H)))
            s *= 2
        tri = jnp.where(ge, fwd, bwd)
        tri_exp = jnp.broadcast_to(tri.reshape(S, 1, S, H),
                                   (S, NSC, S, H)).reshape(S, L, H)
        ctx_groups.append(jnp.where(nsc == scg, tri_exp, base))

    ctx = jnp.concatenate(ctx_groups, axis=0)

    s3 = jnp.dot(ctx.reshape(C * L, H), w3t_ref[:],
                 preferred_element_type=jnp.float32).reshape(C, L, H)

    cn = cn_ref[:]
    half = jnp.float32(0.5)
    isq2 = jnp.float32(0.7071067811865476)
    ones_col = jnp.ones((L, 1), jnp.float32)
    for mi in range(C):
        z_mi = jnp.transpose(zb_ref[pl.ds(j * C + mi, 1)][0], (1, 0))
        z_aug = jnp.concatenate([z_mi, ones_col], axis=1)
        w4aug = jnp.concatenate(
            [w4t_ref[:], a1_ref[pl.ds(j * C + mi, 1)]], axis=0)
        t4_mi = jnp.dot(z_aug, w4aug, preferred_element_type=jnp.float32)
        acc = s3[mi] + cn + t4_mi
        out_ref[0, mi] = half * acc * (1.0 + jax.lax.erf(acc * isq2))


def kernel(x, y, V, W_w, W_b):
    B, L, H = x.shape
    D = V.shape[0]
    BM = _BM

    w1t = W_w[:, :H].T
    w2t = W_w[:, H:2 * H].T
    w3t = W_w[:, 2 * H:3 * H].T
    w4t = W_w[:, 3 * H:].T
    vr = jnp.transpose(V, (1, 0, 2)).reshape(H, D * H)
    wb2 = W_b[None, :]

    return pl.pallas_call(
        _body,
        grid=(B, L // BM),
        compiler_params=pltpu.CompilerParams(
            dimension_semantics=("parallel", "arbitrary")),
        in_specs=[
            pl.BlockSpec((1, L, H), lambda b, j: (b, 0, 0)),
            pl.BlockSpec((1, BM, H), lambda b, j: (b, j, 0)),
            pl.BlockSpec((1, L, H), lambda b, j: (b, 0, 0)),
            pl.BlockSpec((H, H), lambda b, j: (0, 0)),
            pl.BlockSpec((H, H), lambda b, j: (0, 0)),
            pl.BlockSpec((H, H), lambda b, j: (0, 0)),
            pl.BlockSpec((H, D * H), lambda b, j: (0, 0)),
            pl.BlockSpec((D, H), lambda b, j: (0, 0)),
            pl.BlockSpec((1, H), lambda b, j: (0, 0)),
        ],
        out_specs=pl.BlockSpec((1, BM, L, H), lambda b, j: (b, j, 0, 0)),
        out_shape=jax.ShapeDtypeStruct((B, L, L, H), jnp.float32),
        scratch_shapes=[
            pltpu.VMEM((L, H), jnp.float32),
            pltpu.VMEM((L, H), jnp.float32),
            pltpu.VMEM((L, H), jnp.float32),
            pltpu.VMEM((L // _S, H), jnp.float32),
            pltpu.VMEM((L, D, L), jnp.float32),
            pltpu.VMEM((L, H), jnp.float32),
        ],
    )(x, x, y, w1t, w2t, w3t, vr, w4t, wb2)
